# baseline (speedup 1.0000x reference)
import functools

import jax
import jax.numpy as jnp
from jax import lax
from jax.experimental import pallas as pl
from jax.experimental.pallas import tpu as pltpu
from jax.experimental.pallas import tpu_sc as plsc

B = 16384
L = 20
DA = 64
DT = 64

NC = 2
NS = 16
NW = NC * NS
RPW = B // NW
CB = 4
IPC = CB * L
NCHUNK = RPW // CB


NBUF = 2


def _sc_pool_body(artist_ref, title_ref, emb_a_ref, emb_t_ref, out_ref,
                  idx_a, idx_t, rows_a, rows_t, out_v, sems_a, sems_t):
    wid = lax.axis_index("s") * NC + lax.axis_index("c")
    pltpu.sync_copy(artist_ref.at[wid], idx_a)
    pltpu.sync_copy(title_ref.at[wid], idx_t)

    inv_l = jnp.float32(1.0 / L)

    for b in range(NBUF):
        pltpu.async_copy(emb_a_ref.at[idx_a.at[b]], rows_a.at[b], sems_a[b])
        pltpu.async_copy(emb_t_ref.at[idx_t.at[b]], rows_t.at[b], sems_t[b])

    def step(jj, carry):
        for b in range(NBUF):
            j = jj * NBUF + b
            pltpu.make_async_copy(
                emb_a_ref.at[idx_a.at[j]], rows_a.at[b], sems_a[b]).wait()
            pltpu.make_async_copy(
                emb_t_ref.at[idx_t.at[j]], rows_t.at[b], sems_t[b]).wait()
            for r in range(CB):
                row = j * CB + r
                for k in range(DA // 16):
                    sl = pl.ds(k * 16, 16)
                    acc = rows_a[b, L * r, sl]
                    for q in range(1, L):
                        acc = acc + rows_a[b, L * r + q, sl]
                    out_v[row, sl] = acc * inv_l
                for k in range(DT // 16):
                    sl = pl.ds(k * 16, 16)
                    acc = rows_t[b, L * r, sl]
                    for q in range(1, L):
                        acc = acc + rows_t[b, L * r + q, sl]
                    out_v[row, pl.ds(DA + k * 16, 16)] = acc * inv_l
            nxt = j + NBUF

            @pl.when(nxt < NCHUNK)
            def _fire():
                pltpu.async_copy(
                    emb_a_ref.at[idx_a.at[nxt]], rows_a.at[b], sems_a[b])
                pltpu.async_copy(
                    emb_t_ref.at[idx_t.at[nxt]], rows_t.at[b], sems_t[b])
        return carry

    lax.fori_loop(0, NCHUNK // NBUF, step, 0)
    pltpu.sync_copy(out_v, out_ref.at[pl.ds(wid * RPW, RPW)])


@functools.partial(jax.jit, static_argnums=())
def _sc_pool(artist_r, title_r, emb_artist, emb_title):
    mesh = plsc.VectorSubcoreMesh(core_axis_name="c", subcore_axis_name="s",
                                  num_cores=NC, num_subcores=NS)
    return pl.kernel(
        _sc_pool_body,
        out_type=jax.ShapeDtypeStruct((B, DA + DT), jnp.float32),
        mesh=mesh,
        scratch_types=[
            pltpu.VMEM((NCHUNK, IPC), jnp.int32),
            pltpu.VMEM((NCHUNK, IPC), jnp.int32),
            pltpu.VMEM((NBUF, IPC, DA), jnp.float32),
            pltpu.VMEM((NBUF, IPC, DT), jnp.float32),
            pltpu.VMEM((RPW, DA + DT), jnp.float32),
            [pltpu.SemaphoreType.DMA] * NBUF,
            [pltpu.SemaphoreType.DMA] * NBUF,
        ],
        compiler_params=pltpu.CompilerParams(use_tc_tiling_on_sc=False),
    )(artist_r, title_r, emb_artist, emb_title)


def _mlp_body(x_ref, num_ref, w1e_ref, w1n_ref, b1_ref, w2_ref, b2_ref,
              w3t_ref, b3_ref, out_ref):
    x = x_ref[...]
    h1 = jnp.dot(x, w1e_ref[...], preferred_element_type=jnp.float32)
    num = num_ref[...]
    h1 = h1 + num[:, 0:1] * w1n_ref[0:1, :] + num[:, 1:2] * w1n_ref[1:2, :]
    h1 = jnp.maximum(h1 + b1_ref[...], 0.0)
    h2 = jnp.dot(h1, w2_ref[...], preferred_element_type=jnp.float32)
    h2 = jnp.maximum(h2 + b2_ref[...], 0.0)
    out_ref[...] = jnp.sum(h2 * w3t_ref[...], axis=1, keepdims=True) + b3_ref[...]


def _mlp(pooled, num, w1e, w1n, b1, w2, b2, w3t, b3):
    bb = 2048
    grid = (B // bb,)
    return pl.pallas_call(
        _mlp_body,
        grid=grid,
        in_specs=[
            pl.BlockSpec((bb, DA + DT), lambda i: (i, 0)),
            pl.BlockSpec((bb, 2), lambda i: (i, 0)),
            pl.BlockSpec((DA + DT, 128), lambda i: (0, 0)),
            pl.BlockSpec((2, 128), lambda i: (0, 0)),
            pl.BlockSpec((1, 128), lambda i: (0, 0)),
            pl.BlockSpec((128, 64), lambda i: (0, 0)),
            pl.BlockSpec((1, 64), lambda i: (0, 0)),
            pl.BlockSpec((1, 64), lambda i: (0, 0)),
            pl.BlockSpec((1, 1), lambda i: (0, 0)),
        ],
        out_specs=pl.BlockSpec((bb, 1), lambda i: (i, 0)),
        out_shape=jax.ShapeDtypeStruct((B, 1), jnp.float32),
    )(pooled, num, w1e, w1n, b1, w2, b2, w3t, b3)


def kernel(artist, title, numerical_data, emb_artist, emb_title,
           W1, b1, W2, b2, W3, b3):
    artist_r = artist.astype(jnp.int32).reshape(NW, NCHUNK, IPC)
    title_r = title.astype(jnp.int32).reshape(NW, NCHUNK, IPC)
    pooled = _sc_pool(artist_r, title_r, emb_artist, emb_title)
    w1e = W1[: DA + DT]
    w1n = W1[DA + DT:]
    return _mlp(pooled, numerical_data, w1e, w1n, b1.reshape(1, 128),
                W2, b2.reshape(1, 64), W3.reshape(1, 64), b3.reshape(1, 1))


# speedup vs baseline: 1.0573x; 1.0573x over previous
import functools

import jax
import jax.numpy as jnp
from jax import lax
from jax.experimental import pallas as pl
from jax.experimental.pallas import tpu as pltpu
from jax.experimental.pallas import tpu_sc as plsc

B = 16384
L = 20
DA = 64
DT = 64

NC = 2
NS = 16
NW = NC * NS
RPW = B // NW
CB = 4
IPC = CB * L
NCHUNK = RPW // CB


NBUF = 2


def _sc_pool_body(artist_ref, title_ref, emb_a_ref, emb_t_ref, out_ref,
                  idx_a, idx_t, rows_a, rows_t, out_v, sems_a, sems_t):
    wid = lax.axis_index("s") * NC + lax.axis_index("c")
    pltpu.sync_copy(artist_ref.at[wid], idx_a)
    pltpu.sync_copy(title_ref.at[wid], idx_t)

    inv_l = jnp.float32(1.0 / L)

    for b in range(NBUF):
        pltpu.async_copy(emb_a_ref.at[idx_a.at[b]], rows_a.at[b], sems_a[b])
        pltpu.async_copy(emb_t_ref.at[idx_t.at[b]], rows_t.at[b], sems_t[b])

    def step(jj, carry):
        for b in range(NBUF):
            j = jj * NBUF + b
            pltpu.make_async_copy(
                emb_a_ref.at[idx_a.at[j]], rows_a.at[b], sems_a[b]).wait()
            pltpu.make_async_copy(
                emb_t_ref.at[idx_t.at[j]], rows_t.at[b], sems_t[b]).wait()
            for r in range(CB):
                row = j * CB + r
                for k in range(DA // 16):
                    sl = pl.ds(k * 16, 16)
                    acc = rows_a[b, L * r, sl]
                    for q in range(1, L):
                        acc = acc + rows_a[b, L * r + q, sl]
                    out_v[row, sl] = acc * inv_l
                for k in range(DT // 16):
                    sl = pl.ds(k * 16, 16)
                    acc = rows_t[b, L * r, sl]
                    for q in range(1, L):
                        acc = acc + rows_t[b, L * r + q, sl]
                    out_v[row, pl.ds(DA + k * 16, 16)] = acc * inv_l
            nxt = j + NBUF

            @pl.when(nxt < NCHUNK)
            def _fire():
                pltpu.async_copy(
                    emb_a_ref.at[idx_a.at[nxt]], rows_a.at[b], sems_a[b])
                pltpu.async_copy(
                    emb_t_ref.at[idx_t.at[nxt]], rows_t.at[b], sems_t[b])
        return carry

    lax.fori_loop(0, NCHUNK // NBUF, step, 0)
    pltpu.sync_copy(out_v, out_ref.at[pl.ds(wid * RPW, RPW)])


@functools.partial(jax.jit, static_argnums=())
def _sc_pool(artist_r, title_r, emb_artist, emb_title):
    mesh = plsc.VectorSubcoreMesh(core_axis_name="c", subcore_axis_name="s",
                                  num_cores=NC, num_subcores=NS)
    return pl.kernel(
        _sc_pool_body,
        out_type=jax.ShapeDtypeStruct((B, DA + DT), jnp.float32),
        mesh=mesh,
        scratch_types=[
            pltpu.VMEM((NCHUNK, IPC), jnp.int32),
            pltpu.VMEM((NCHUNK, IPC), jnp.int32),
            pltpu.VMEM((NBUF, IPC, 128), jnp.float32),
            pltpu.VMEM((NBUF, IPC, 128), jnp.float32),
            pltpu.VMEM((RPW, DA + DT), jnp.float32),
            [pltpu.SemaphoreType.DMA] * NBUF,
            [pltpu.SemaphoreType.DMA] * NBUF,
        ],
        compiler_params=pltpu.CompilerParams(use_tc_tiling_on_sc=False),
    )(artist_r, title_r, emb_artist, emb_title)


def _mlp_body(x_ref, num_ref, w1e_ref, w1n_ref, b1_ref, w2_ref, b2_ref,
              w3t_ref, b3_ref, out_ref):
    x = x_ref[...]
    h1 = jnp.dot(x, w1e_ref[...], preferred_element_type=jnp.float32)
    num = num_ref[...]
    h1 = h1 + num[:, 0:1] * w1n_ref[0:1, :] + num[:, 1:2] * w1n_ref[1:2, :]
    h1 = jnp.maximum(h1 + b1_ref[...], 0.0)
    h2 = jnp.dot(h1, w2_ref[...], preferred_element_type=jnp.float32)
    h2 = jnp.maximum(h2 + b2_ref[...], 0.0)
    out_ref[...] = jnp.sum(h2 * w3t_ref[...], axis=1, keepdims=True) + b3_ref[...]


def _mlp(pooled, num, w1e, w1n, b1, w2, b2, w3t, b3):
    bb = 2048
    grid = (B // bb,)
    return pl.pallas_call(
        _mlp_body,
        grid=grid,
        in_specs=[
            pl.BlockSpec((bb, DA + DT), lambda i: (i, 0)),
            pl.BlockSpec((bb, 2), lambda i: (i, 0)),
            pl.BlockSpec((DA + DT, 128), lambda i: (0, 0)),
            pl.BlockSpec((2, 128), lambda i: (0, 0)),
            pl.BlockSpec((1, 128), lambda i: (0, 0)),
            pl.BlockSpec((128, 64), lambda i: (0, 0)),
            pl.BlockSpec((1, 64), lambda i: (0, 0)),
            pl.BlockSpec((1, 64), lambda i: (0, 0)),
            pl.BlockSpec((1, 1), lambda i: (0, 0)),
        ],
        out_specs=pl.BlockSpec((bb, 1), lambda i: (i, 0)),
        out_shape=jax.ShapeDtypeStruct((B, 1), jnp.float32),
    )(pooled, num, w1e, w1n, b1, w2, b2, w3t, b3)


def kernel(artist, title, numerical_data, emb_artist, emb_title,
           W1, b1, W2, b2, W3, b3):
    artist_r = artist.astype(jnp.int32).reshape(NW, NCHUNK, IPC)
    title_r = title.astype(jnp.int32).reshape(NW, NCHUNK, IPC)
    at128 = jnp.pad(emb_artist, ((0, 0), (0, 128 - DA)))
    tt128 = jnp.pad(emb_title, ((0, 0), (0, 128 - DT)))
    pooled = _sc_pool(artist_r, title_r, at128, tt128)
    w1e = W1[: DA + DT]
    w1n = W1[DA + DT:]
    return _mlp(pooled, numerical_data, w1e, w1n, b1.reshape(1, 128),
                W2, b2.reshape(1, 64), W3.reshape(1, 64), b3.reshape(1, 1))


# speedup vs baseline: 1.0960x; 1.0367x over previous
import functools

import jax
import jax.numpy as jnp
from jax import lax
from jax.experimental import pallas as pl
from jax.experimental.pallas import tpu as pltpu
from jax.experimental.pallas import tpu_sc as plsc

B = 16384
L = 20
DA = 64
DT = 64

NC = 2
NS = 16
NW = NC * NS
RPW = B // NW
CB = 4
IPC = CB * L
NCHUNK = RPW // CB


NBUF = 2


def _sc_pool_body(artist_ref, title_ref, emb_a_ref, emb_t_ref, out_ref,
                  idx_a, idx_t, rows_a, rows_t, out_v, sems_a, sems_t):
    wid = lax.axis_index("s") * NC + lax.axis_index("c")
    pltpu.sync_copy(artist_ref.at[wid], idx_a)
    pltpu.sync_copy(title_ref.at[wid], idx_t)

    inv_l = jnp.float32(1.0 / L)

    for b in range(NBUF):
        pltpu.async_copy(emb_a_ref.at[idx_a.at[b]], rows_a.at[b], sems_a[b])
        pltpu.async_copy(emb_t_ref.at[idx_t.at[b]], rows_t.at[b], sems_t[b])

    def step(jj, carry):
        for b in range(NBUF):
            j = jj * NBUF + b
            pltpu.make_async_copy(
                emb_a_ref.at[idx_a.at[j]], rows_a.at[b], sems_a[b]).wait()
            pltpu.make_async_copy(
                emb_t_ref.at[idx_t.at[j]], rows_t.at[b], sems_t[b]).wait()
            for r in range(CB):
                row = j * CB + r
                for k in range(DA // 16):
                    sl = pl.ds(k * 16, 16)
                    acc = rows_a[b, L * r, sl]
                    for q in range(1, L):
                        acc = acc + rows_a[b, L * r + q, sl]
                    out_v[row, sl] = acc * inv_l
                for k in range(DT // 16):
                    sl = pl.ds(k * 16, 16)
                    acc = rows_t[b, L * r, sl]
                    for q in range(1, L):
                        acc = acc + rows_t[b, L * r + q, sl]
                    out_v[row, pl.ds(DA + k * 16, 16)] = acc * inv_l
            nxt = j + NBUF

            @pl.when(nxt < NCHUNK)
            def _fire():
                pltpu.async_copy(
                    emb_a_ref.at[idx_a.at[nxt]], rows_a.at[b], sems_a[b])
                pltpu.async_copy(
                    emb_t_ref.at[idx_t.at[nxt]], rows_t.at[b], sems_t[b])
        return carry

    lax.fori_loop(0, NCHUNK // NBUF, step, 0)
    pltpu.sync_copy(out_v, out_ref.at[pl.ds(wid * RPW, RPW)])


@functools.partial(jax.jit, static_argnums=())
def _sc_pool(artist_r, title_r, emb_artist, emb_title):
    mesh = plsc.VectorSubcoreMesh(core_axis_name="c", subcore_axis_name="s",
                                  num_cores=NC, num_subcores=NS)
    return pl.kernel(
        _sc_pool_body,
        out_type=jax.ShapeDtypeStruct((B, DA + DT), jnp.float32),
        mesh=mesh,
        scratch_types=[
            pltpu.VMEM((NCHUNK, IPC), jnp.int32),
            pltpu.VMEM((NCHUNK, IPC), jnp.int32),
            pltpu.VMEM((NBUF, IPC, 128), jnp.float32),
            pltpu.VMEM((NBUF, IPC, 128), jnp.float32),
            pltpu.VMEM((RPW, DA + DT), jnp.float32),
            [pltpu.SemaphoreType.DMA] * NBUF,
            [pltpu.SemaphoreType.DMA] * NBUF,
        ],
        compiler_params=pltpu.CompilerParams(use_tc_tiling_on_sc=False),
    )(artist_r, title_r, emb_artist, emb_title)


def _tr_body(in_ref, out_ref):
    x = in_ref[...]
    out_ref[...] = jnp.concatenate(
        [x.T, jnp.zeros((x.shape[1], 128 - x.shape[0]), jnp.float32)], axis=1)


def _transpose_pad(table_t, v):
    tb = 2048
    grid = (pl.cdiv(v, tb),)
    return pl.pallas_call(
        _tr_body,
        grid=grid,
        in_specs=[pl.BlockSpec((64, tb), lambda i: (0, i))],
        out_specs=pl.BlockSpec((tb, 128), lambda i: (i, 0)),
        out_shape=jax.ShapeDtypeStruct((v, 128), jnp.float32),
    )(table_t)


def _mlp_body(x_ref, num_ref, w1e_ref, w1n_ref, b1_ref, w2_ref, b2_ref,
              w3t_ref, b3_ref, out_ref):
    x = x_ref[...]
    h1 = jnp.dot(x, w1e_ref[...], preferred_element_type=jnp.float32,
                 precision=jax.lax.Precision.HIGHEST)
    num = num_ref[...]
    h1 = h1 + num[:, 0:1] * w1n_ref[0:1, :] + num[:, 1:2] * w1n_ref[1:2, :]
    h1 = jnp.maximum(h1 + b1_ref[...], 0.0)
    h2 = jnp.dot(h1, w2_ref[...], preferred_element_type=jnp.float32,
                 precision=jax.lax.Precision.HIGHEST)
    h2 = jnp.maximum(h2 + b2_ref[...], 0.0)
    out_ref[...] = jnp.sum(h2 * w3t_ref[...], axis=1, keepdims=True) + b3_ref[...]


def _mlp(pooled, num, w1e, w1n, b1, w2, b2, w3t, b3):
    bb = 2048
    grid = (B // bb,)
    return pl.pallas_call(
        _mlp_body,
        grid=grid,
        in_specs=[
            pl.BlockSpec((bb, DA + DT), lambda i: (i, 0)),
            pl.BlockSpec((bb, 2), lambda i: (i, 0)),
            pl.BlockSpec((DA + DT, 128), lambda i: (0, 0)),
            pl.BlockSpec((2, 128), lambda i: (0, 0)),
            pl.BlockSpec((1, 128), lambda i: (0, 0)),
            pl.BlockSpec((128, 64), lambda i: (0, 0)),
            pl.BlockSpec((1, 64), lambda i: (0, 0)),
            pl.BlockSpec((1, 64), lambda i: (0, 0)),
            pl.BlockSpec((1, 1), lambda i: (0, 0)),
        ],
        out_specs=pl.BlockSpec((bb, 1), lambda i: (i, 0)),
        out_shape=jax.ShapeDtypeStruct((B, 1), jnp.float32),
    )(pooled, num, w1e, w1n, b1, w2, b2, w3t, b3)


def kernel(artist, title, numerical_data, emb_artist, emb_title,
           W1, b1, W2, b2, W3, b3):
    artist_r = artist.astype(jnp.int32).reshape(NW, NCHUNK, IPC)
    title_r = title.astype(jnp.int32).reshape(NW, NCHUNK, IPC)
    at128 = _transpose_pad(emb_artist.T, emb_artist.shape[0])
    tt128 = _transpose_pad(emb_title.T, emb_title.shape[0])
    pooled = _sc_pool(artist_r, title_r, at128, tt128)
    w1e = W1[: DA + DT]
    w1n = W1[DA + DT:]
    return _mlp(pooled, numerical_data, w1e, w1n, b1.reshape(1, 128),
                W2, b2.reshape(1, 64), W3.reshape(1, 64), b3.reshape(1, 1))


# speedup vs baseline: 1.4475x; 1.3207x over previous
import functools

import jax
import jax.numpy as jnp
from jax import lax
from jax.experimental import pallas as pl
from jax.experimental.pallas import tpu as pltpu
from jax.experimental.pallas import tpu_sc as plsc

B = 16384
L = 20
D = 64

NC = 2
NS = 16
NW = NC * NS
RPW = B // NW
CB = 4
IPC = CB * L
NCHUNK = RPW // CB
NBUF = 2


def _sc_pool_body(idx_ref, emb_ref, out_ref, idx_v, rows, out_v, sems):
    wid = lax.axis_index("s") * NC + lax.axis_index("c")
    pltpu.sync_copy(idx_ref.at[wid], idx_v)

    inv_l = jnp.float32(1.0 / L)

    for b in range(NBUF):
        pltpu.async_copy(emb_ref.at[idx_v.at[b]], rows.at[b], sems[b])

    def step(jj, carry):
        for b in range(NBUF):
            j = jj * NBUF + b
            pltpu.make_async_copy(
                emb_ref.at[idx_v.at[j]], rows.at[b], sems[b]).wait()
            for r in range(CB):
                row = j * CB + r
                for k in range(D // 16):
                    sl = pl.ds(k * 16, 16)
                    acc = rows[b, L * r, sl]
                    for q in range(1, L):
                        acc = acc + rows[b, L * r + q, sl]
                    out_v[row, sl] = acc * inv_l
            nxt = j + NBUF

            @pl.when(nxt < NCHUNK)
            def _fire():
                pltpu.async_copy(emb_ref.at[idx_v.at[nxt]], rows.at[b], sems[b])
        return carry

    lax.fori_loop(0, NCHUNK // NBUF, step, 0)
    pltpu.sync_copy(out_v, out_ref.at[pl.ds(wid * RPW, RPW)])


def _sc_pool(idx_r, table128):
    mesh = plsc.VectorSubcoreMesh(core_axis_name="c", subcore_axis_name="s",
                                  num_cores=NC, num_subcores=NS)
    return pl.kernel(
        _sc_pool_body,
        out_type=jax.ShapeDtypeStruct((B, D), jnp.float32),
        mesh=mesh,
        scratch_types=[
            pltpu.VMEM((NCHUNK, IPC), jnp.int32),
            pltpu.VMEM((NBUF, IPC, 128), jnp.float32),
            pltpu.VMEM((RPW, D), jnp.float32),
            [pltpu.SemaphoreType.DMA] * NBUF,
        ],
        compiler_params=pltpu.CompilerParams(use_tc_tiling_on_sc=False),
    )(idx_r, table128)


def _tr_body(in_ref, out_ref):
    x = in_ref[...]
    out_ref[...] = jnp.concatenate(
        [x.T, jnp.zeros((x.shape[1], 128 - x.shape[0]), jnp.float32)], axis=1)


def _transpose_pad(table_t, v):
    tb = 4096
    grid = (pl.cdiv(v, tb),)
    return pl.pallas_call(
        _tr_body,
        grid=grid,
        in_specs=[pl.BlockSpec((64, tb), lambda i: (0, i))],
        out_specs=pl.BlockSpec((tb, 128), lambda i: (i, 0)),
        out_shape=jax.ShapeDtypeStruct((v, 128), jnp.float32),
    )(table_t)


def _mlp_body(xa_ref, xt_ref, num_ref, w1a_ref, w1t_ref, w1n_ref, b1_ref,
              w2_ref, b2_ref, w3t_ref, b3_ref, out_ref):
    h1 = jnp.dot(xa_ref[...], w1a_ref[...], preferred_element_type=jnp.float32,
                 precision=jax.lax.Precision.HIGHEST)
    h1 = h1 + jnp.dot(xt_ref[...], w1t_ref[...],
                      preferred_element_type=jnp.float32,
                      precision=jax.lax.Precision.HIGHEST)
    num = num_ref[...]
    h1 = h1 + num[:, 0:1] * w1n_ref[0:1, :] + num[:, 1:2] * w1n_ref[1:2, :]
    h1 = jnp.maximum(h1 + b1_ref[...], 0.0)
    h2 = jnp.dot(h1, w2_ref[...], preferred_element_type=jnp.float32,
                 precision=jax.lax.Precision.HIGHEST)
    h2 = jnp.maximum(h2 + b2_ref[...], 0.0)
    out_ref[...] = jnp.sum(h2 * w3t_ref[...], axis=1, keepdims=True) + b3_ref[...]


def _mlp(xa, xt, num, w1a, w1t, w1n, b1, w2, b2, w3t, b3):
    bb = 2048
    grid = (B // bb,)
    return pl.pallas_call(
        _mlp_body,
        grid=grid,
        in_specs=[
            pl.BlockSpec((bb, D), lambda i: (i, 0)),
            pl.BlockSpec((bb, D), lambda i: (i, 0)),
            pl.BlockSpec((bb, 2), lambda i: (i, 0)),
            pl.BlockSpec((D, 128), lambda i: (0, 0)),
            pl.BlockSpec((D, 128), lambda i: (0, 0)),
            pl.BlockSpec((2, 128), lambda i: (0, 0)),
            pl.BlockSpec((1, 128), lambda i: (0, 0)),
            pl.BlockSpec((128, 64), lambda i: (0, 0)),
            pl.BlockSpec((1, 64), lambda i: (0, 0)),
            pl.BlockSpec((1, 64), lambda i: (0, 0)),
            pl.BlockSpec((1, 1), lambda i: (0, 0)),
        ],
        out_specs=pl.BlockSpec((bb, 1), lambda i: (i, 0)),
        out_shape=jax.ShapeDtypeStruct((B, 1), jnp.float32),
    )(xa, xt, num, w1a, w1t, w1n, b1, w2, b2, w3t, b3)


def kernel(artist, title, numerical_data, emb_artist, emb_title,
           W1, b1, W2, b2, W3, b3):
    artist_r = artist.astype(jnp.int32).reshape(NW, NCHUNK, IPC)
    title_r = title.astype(jnp.int32).reshape(NW, NCHUNK, IPC)
    tt128 = _transpose_pad(emb_title.T, emb_title.shape[0])
    pooled_t = _sc_pool(title_r, tt128)
    at128 = _transpose_pad(emb_artist.T, emb_artist.shape[0])
    pooled_a = _sc_pool(artist_r, at128)
    return _mlp(pooled_a, pooled_t, numerical_data,
                W1[:D], W1[D:2 * D], W1[2 * D:], b1.reshape(1, 128),
                W2, b2.reshape(1, 64), W3.reshape(1, 64), b3.reshape(1, 1))


# speedup vs baseline: 1.4991x; 1.0356x over previous
import functools

import jax
import jax.numpy as jnp
from jax import lax
from jax.experimental import pallas as pl
from jax.experimental.pallas import tpu as pltpu
from jax.experimental.pallas import tpu_sc as plsc

B = 16384
L = 20
D = 64

NC = 2
NS = 16
NW = NC * NS
RPW = B // NW
CB = 4
IPC = CB * L
NCHUNK = RPW // CB
NBUF = 4


def _sc_pool_body(idx_ref, emb_ref, out_ref, idx_v, rows, out_v, sems):
    wid = lax.axis_index("s") * NC + lax.axis_index("c")
    pltpu.sync_copy(idx_ref.at[wid], idx_v)

    inv_l = jnp.float32(1.0 / L)

    for b in range(NBUF):
        pltpu.async_copy(emb_ref.at[idx_v.at[b]], rows.at[b], sems[b])

    def step(jj, carry):
        for b in range(NBUF):
            j = jj * NBUF + b
            pltpu.make_async_copy(
                emb_ref.at[idx_v.at[j]], rows.at[b], sems[b]).wait()
            for r in range(CB):
                row = j * CB + r
                for k in range(D // 16):
                    sl = pl.ds(k * 16, 16)
                    acc = rows[b, L * r, sl]
                    for q in range(1, L):
                        acc = acc + rows[b, L * r + q, sl]
                    out_v[row, sl] = acc * inv_l
            nxt = j + NBUF

            @pl.when(nxt < NCHUNK)
            def _fire():
                pltpu.async_copy(emb_ref.at[idx_v.at[nxt]], rows.at[b], sems[b])
        return carry

    lax.fori_loop(0, NCHUNK // NBUF, step, 0)
    pltpu.sync_copy(out_v, out_ref.at[pl.ds(wid * RPW, RPW)])


def _sc_pool(idx_r, table128):
    mesh = plsc.VectorSubcoreMesh(core_axis_name="c", subcore_axis_name="s",
                                  num_cores=NC, num_subcores=NS)
    return pl.kernel(
        _sc_pool_body,
        out_type=jax.ShapeDtypeStruct((B, D), jnp.float32),
        mesh=mesh,
        scratch_types=[
            pltpu.VMEM((NCHUNK, IPC), jnp.int32),
            pltpu.VMEM((NBUF, IPC, 128), jnp.float32),
            pltpu.VMEM((RPW, D), jnp.float32),
            [pltpu.SemaphoreType.DMA] * NBUF,
        ],
        compiler_params=pltpu.CompilerParams(use_tc_tiling_on_sc=False),
    )(idx_r, table128)


def _tr_body(in_ref, out_ref):
    x = in_ref[...]
    out_ref[...] = jnp.concatenate(
        [x.T, jnp.zeros((x.shape[1], 128 - x.shape[0]), jnp.float32)], axis=1)


def _transpose_pad(table_t, v):
    tb = 4096
    grid = (pl.cdiv(v, tb),)
    return pl.pallas_call(
        _tr_body,
        grid=grid,
        in_specs=[pl.BlockSpec((64, tb), lambda i: (0, i))],
        out_specs=pl.BlockSpec((tb, 128), lambda i: (i, 0)),
        out_shape=jax.ShapeDtypeStruct((v, 128), jnp.float32),
    )(table_t)


def _mlp_body(xa_ref, xt_ref, num_ref, w1a_ref, w1t_ref, w1n_ref, b1_ref,
              w2_ref, b2_ref, w3t_ref, b3_ref, out_ref):
    h1 = jnp.dot(xa_ref[...], w1a_ref[...], preferred_element_type=jnp.float32,
                 precision=jax.lax.Precision.HIGHEST)
    h1 = h1 + jnp.dot(xt_ref[...], w1t_ref[...],
                      preferred_element_type=jnp.float32,
                      precision=jax.lax.Precision.HIGHEST)
    num = num_ref[...]
    h1 = h1 + num[:, 0:1] * w1n_ref[0:1, :] + num[:, 1:2] * w1n_ref[1:2, :]
    h1 = jnp.maximum(h1 + b1_ref[...], 0.0)
    h2 = jnp.dot(h1, w2_ref[...], preferred_element_type=jnp.float32,
                 precision=jax.lax.Precision.HIGHEST)
    h2 = jnp.maximum(h2 + b2_ref[...], 0.0)
    out_ref[...] = jnp.sum(h2 * w3t_ref[...], axis=1, keepdims=True) + b3_ref[...]


def _mlp(xa, xt, num, w1a, w1t, w1n, b1, w2, b2, w3t, b3):
    bb = 2048
    grid = (B // bb,)
    return pl.pallas_call(
        _mlp_body,
        grid=grid,
        in_specs=[
            pl.BlockSpec((bb, D), lambda i: (i, 0)),
            pl.BlockSpec((bb, D), lambda i: (i, 0)),
            pl.BlockSpec((bb, 2), lambda i: (i, 0)),
            pl.BlockSpec((D, 128), lambda i: (0, 0)),
            pl.BlockSpec((D, 128), lambda i: (0, 0)),
            pl.BlockSpec((2, 128), lambda i: (0, 0)),
            pl.BlockSpec((1, 128), lambda i: (0, 0)),
            pl.BlockSpec((128, 64), lambda i: (0, 0)),
            pl.BlockSpec((1, 64), lambda i: (0, 0)),
            pl.BlockSpec((1, 64), lambda i: (0, 0)),
            pl.BlockSpec((1, 1), lambda i: (0, 0)),
        ],
        out_specs=pl.BlockSpec((bb, 1), lambda i: (i, 0)),
        out_shape=jax.ShapeDtypeStruct((B, 1), jnp.float32),
    )(xa, xt, num, w1a, w1t, w1n, b1, w2, b2, w3t, b3)


def kernel(artist, title, numerical_data, emb_artist, emb_title,
           W1, b1, W2, b2, W3, b3):
    artist_r = artist.astype(jnp.int32).reshape(NW, NCHUNK, IPC)
    title_r = title.astype(jnp.int32).reshape(NW, NCHUNK, IPC)
    tt128 = _transpose_pad(emb_title.T, emb_title.shape[0])
    at_view, tt128 = lax.optimization_barrier((emb_artist.T, tt128))
    pooled_t = _sc_pool(title_r, tt128)
    at128 = _transpose_pad(at_view, emb_artist.shape[0])
    pooled_a = _sc_pool(artist_r, at128)
    return _mlp(pooled_a, pooled_t, numerical_data,
                W1[:D], W1[D:2 * D], W1[2 * D:], b1.reshape(1, 128),
                W2, b2.reshape(1, 64), W3.reshape(1, 64), b3.reshape(1, 1))


# speedup vs baseline: 1.5873x; 1.0588x over previous
import functools

import jax
import jax.numpy as jnp
from jax import lax
from jax.experimental import pallas as pl
from jax.experimental.pallas import tpu as pltpu
from jax.experimental.pallas import tpu_sc as plsc

B = 16384
L = 20
D = 64

NC = 2
NS = 16
NW = NC * NS
RPW = B // NW
CB = 4
IPC = CB * L
NCHUNK = RPW // CB
NBUF = 4


def _sc_pool_body(idx_ref, emb_ref, out_ref, idx_v, rows, out_v, sems):
    wid = lax.axis_index("s") * NC + lax.axis_index("c")
    pltpu.sync_copy(idx_ref.at[wid], idx_v)

    inv_l = jnp.float32(1.0 / L)

    for b in range(NBUF):
        pltpu.async_copy(emb_ref.at[idx_v.at[b]], rows.at[b], sems[b])

    def step(jj, carry):
        for b in range(NBUF):
            j = jj * NBUF + b
            pltpu.make_async_copy(
                emb_ref.at[idx_v.at[j]], rows.at[b], sems[b]).wait()
            for r in range(CB):
                row = j * CB + r
                for k in range(D // 16):
                    sl = pl.ds(k * 16, 16)
                    acc = rows[b, L * r, sl]
                    for q in range(1, L):
                        acc = acc + rows[b, L * r + q, sl]
                    out_v[row, sl] = acc * inv_l
            nxt = j + NBUF

            @pl.when(nxt < NCHUNK)
            def _fire():
                pltpu.async_copy(emb_ref.at[idx_v.at[nxt]], rows.at[b], sems[b])
        return carry

    lax.fori_loop(0, NCHUNK // NBUF, step, 0)
    pltpu.sync_copy(out_v, out_ref.at[pl.ds(wid * RPW, RPW)])


def _sc_pool(idx_r, table128):
    mesh = plsc.VectorSubcoreMesh(core_axis_name="c", subcore_axis_name="s",
                                  num_cores=NC, num_subcores=NS)
    return pl.kernel(
        _sc_pool_body,
        out_type=jax.ShapeDtypeStruct((B, D), jnp.float32),
        mesh=mesh,
        scratch_types=[
            pltpu.VMEM((NCHUNK, IPC), jnp.int32),
            pltpu.VMEM((NBUF, IPC, D), jnp.float32),
            pltpu.VMEM((RPW, D), jnp.float32),
            [pltpu.SemaphoreType.DMA] * NBUF,
        ],
        compiler_params=pltpu.CompilerParams(use_tc_tiling_on_sc=False),
    )(idx_r, table128)


def _tr_body(a_ref, b_ref, out_ref):
    out_ref[...] = jnp.concatenate([a_ref[...].T, b_ref[...].T], axis=1)


TB2 = 2048


def _repack_k(v):
    return TB2 * pl.cdiv(v // 2, TB2)


def _repack(table_t, v):
    k = _repack_k(v)
    nb = k // TB2
    grid = (nb,)
    bmax = pl.cdiv(v, TB2) - 1
    return pl.pallas_call(
        _tr_body,
        grid=grid,
        in_specs=[
            pl.BlockSpec((64, TB2), lambda i: (0, i)),
            pl.BlockSpec((64, TB2), lambda i: (0, jnp.minimum(i + nb, bmax))),
        ],
        out_specs=pl.BlockSpec((TB2, 128), lambda i: (i, 0)),
        out_shape=jax.ShapeDtypeStruct((k, 128), jnp.float32),
    )(table_t, table_t)


def _mlp_body(xa_ref, xt_ref, num_ref, w1a_ref, w1t_ref, w1n_ref, b1_ref,
              w2_ref, b2_ref, w3t_ref, b3_ref, out_ref):
    h1 = jnp.dot(xa_ref[...], w1a_ref[...], preferred_element_type=jnp.float32,
                 precision=jax.lax.Precision.HIGHEST)
    h1 = h1 + jnp.dot(xt_ref[...], w1t_ref[...],
                      preferred_element_type=jnp.float32,
                      precision=jax.lax.Precision.HIGHEST)
    num = num_ref[...]
    h1 = h1 + num[:, 0:1] * w1n_ref[0:1, :] + num[:, 1:2] * w1n_ref[1:2, :]
    h1 = jnp.maximum(h1 + b1_ref[...], 0.0)
    h2 = jnp.dot(h1, w2_ref[...], preferred_element_type=jnp.float32,
                 precision=jax.lax.Precision.HIGHEST)
    h2 = jnp.maximum(h2 + b2_ref[...], 0.0)
    out_ref[...] = jnp.sum(h2 * w3t_ref[...], axis=1, keepdims=True) + b3_ref[...]


def _mlp(xa, xt, num, w1a, w1t, w1n, b1, w2, b2, w3t, b3):
    bb = 2048
    grid = (B // bb,)
    return pl.pallas_call(
        _mlp_body,
        grid=grid,
        in_specs=[
            pl.BlockSpec((bb, D), lambda i: (i, 0)),
            pl.BlockSpec((bb, D), lambda i: (i, 0)),
            pl.BlockSpec((bb, 2), lambda i: (i, 0)),
            pl.BlockSpec((D, 128), lambda i: (0, 0)),
            pl.BlockSpec((D, 128), lambda i: (0, 0)),
            pl.BlockSpec((2, 128), lambda i: (0, 0)),
            pl.BlockSpec((1, 128), lambda i: (0, 0)),
            pl.BlockSpec((128, 64), lambda i: (0, 0)),
            pl.BlockSpec((1, 64), lambda i: (0, 0)),
            pl.BlockSpec((1, 64), lambda i: (0, 0)),
            pl.BlockSpec((1, 1), lambda i: (0, 0)),
        ],
        out_specs=pl.BlockSpec((bb, 1), lambda i: (i, 0)),
        out_shape=jax.ShapeDtypeStruct((B, 1), jnp.float32),
    )(xa, xt, num, w1a, w1t, w1n, b1, w2, b2, w3t, b3)


def kernel(artist, title, numerical_data, emb_artist, emb_title,
           W1, b1, W2, b2, W3, b3):
    vt = emb_title.shape[0]
    va = emb_artist.shape[0]
    ka = _repack_k(va)
    kt = _repack_k(vt)
    ai = artist.astype(jnp.int32)
    ti = title.astype(jnp.int32)
    artist_r = jnp.where(ai < ka, 2 * ai, 2 * (ai - ka) + 1).reshape(
        NW, NCHUNK, IPC)
    title_r = jnp.where(ti < kt, 2 * ti, 2 * (ti - kt) + 1).reshape(
        NW, NCHUNK, IPC)
    ttp = _repack(emb_title.T, vt)
    at_view, ttp = lax.optimization_barrier((emb_artist.T, ttp))
    pooled_t = _sc_pool(title_r, ttp.reshape(2 * kt, D))
    at64 = _repack(at_view, va).reshape(2 * ka, D)
    pooled_a = _sc_pool(artist_r, at64)
    return _mlp(pooled_a, pooled_t, numerical_data,
                W1[:D], W1[D:2 * D], W1[2 * D:], b1.reshape(1, 128),
                W2, b2.reshape(1, 64), W3.reshape(1, 64), b3.reshape(1, 1))


# speedup vs baseline: 1.7167x; 1.0815x over previous
import functools

import jax
import jax.numpy as jnp
from jax import lax
from jax.experimental import pallas as pl
from jax.experimental.pallas import tpu as pltpu
from jax.experimental.pallas import tpu_sc as plsc

B = 16384
L = 20
D = 64

NC = 2
NS = 16
NW = NC * NS
RPW = B // NW
CB = 4
IPC = CB * L
NCHUNK = RPW // CB
NBUF = 8


def _sc_pool_body(idx_ref, emb_ref, out_ref, idx_v, rows, out_v, sems):
    wid = lax.axis_index("s") * NC + lax.axis_index("c")
    pltpu.sync_copy(idx_ref.at[wid], idx_v)

    inv_l = jnp.float32(1.0 / L)

    for b in range(NBUF):
        pltpu.async_copy(emb_ref.at[idx_v.at[b]], rows.at[b], sems[b])

    def step(jj, carry):
        for b in range(NBUF):
            j = jj * NBUF + b
            pltpu.make_async_copy(
                emb_ref.at[idx_v.at[j]], rows.at[b], sems[b]).wait()
            for r in range(CB):
                row = j * CB + r
                for k in range(D // 16):
                    sl = pl.ds(k * 16, 16)
                    acc = rows[b, L * r, sl]
                    for q in range(1, L):
                        acc = acc + rows[b, L * r + q, sl]
                    out_v[row, sl] = acc * inv_l
            nxt = j + NBUF

            @pl.when(nxt < NCHUNK)
            def _fire():
                pltpu.async_copy(emb_ref.at[idx_v.at[nxt]], rows.at[b], sems[b])
        return carry

    lax.fori_loop(0, NCHUNK // NBUF, step, 0)
    pltpu.sync_copy(out_v, out_ref.at[pl.ds(wid * RPW, RPW)])


def _sc_pool(idx_r, table128):
    mesh = plsc.VectorSubcoreMesh(core_axis_name="c", subcore_axis_name="s",
                                  num_cores=NC, num_subcores=NS)
    return pl.kernel(
        _sc_pool_body,
        out_type=jax.ShapeDtypeStruct((B, D), jnp.float32),
        mesh=mesh,
        scratch_types=[
            pltpu.VMEM((NCHUNK, IPC), jnp.int32),
            pltpu.VMEM((NBUF, IPC, D), jnp.float32),
            pltpu.VMEM((RPW, D), jnp.float32),
            [pltpu.SemaphoreType.DMA] * NBUF,
        ],
        compiler_params=pltpu.CompilerParams(use_tc_tiling_on_sc=False),
    )(idx_r, table128)


def _tr_body(a_ref, b_ref, out_ref):
    out_ref[...] = jnp.concatenate([a_ref[...].T, b_ref[...].T], axis=1)


TB2 = 4096


def _repack_k(v):
    return TB2 * pl.cdiv(v // 2, TB2)


def _repack(table_t, v):
    k = _repack_k(v)
    nb = k // TB2
    grid = (nb,)
    bmax = pl.cdiv(v, TB2) - 1
    return pl.pallas_call(
        _tr_body,
        grid=grid,
        in_specs=[
            pl.BlockSpec((64, TB2), lambda i: (0, i)),
            pl.BlockSpec((64, TB2), lambda i: (0, jnp.minimum(i + nb, bmax))),
        ],
        out_specs=pl.BlockSpec((TB2, 128), lambda i: (i, 0)),
        out_shape=jax.ShapeDtypeStruct((k, 128), jnp.float32),
    )(table_t, table_t)


def _mlp_body(xa_ref, xt_ref, num_ref, w1a_ref, w1t_ref, w1n_ref, b1_ref,
              w2_ref, b2_ref, w3t_ref, b3_ref, out_ref):
    h1 = jnp.dot(xa_ref[...], w1a_ref[...], preferred_element_type=jnp.float32,
                 precision=jax.lax.Precision.HIGHEST)
    h1 = h1 + jnp.dot(xt_ref[...], w1t_ref[...],
                      preferred_element_type=jnp.float32,
                      precision=jax.lax.Precision.HIGHEST)
    num = num_ref[...]
    h1 = h1 + num[:, 0:1] * w1n_ref[0:1, :] + num[:, 1:2] * w1n_ref[1:2, :]
    h1 = jnp.maximum(h1 + b1_ref[...], 0.0)
    h2 = jnp.dot(h1, w2_ref[...], preferred_element_type=jnp.float32,
                 precision=jax.lax.Precision.HIGHEST)
    h2 = jnp.maximum(h2 + b2_ref[...], 0.0)
    out_ref[...] = jnp.sum(h2 * w3t_ref[...], axis=1, keepdims=True) + b3_ref[...]


def _mlp(xa, xt, num, w1a, w1t, w1n, b1, w2, b2, w3t, b3):
    bb = 2048
    grid = (B // bb,)
    return pl.pallas_call(
        _mlp_body,
        grid=grid,
        in_specs=[
            pl.BlockSpec((bb, D), lambda i: (i, 0)),
            pl.BlockSpec((bb, D), lambda i: (i, 0)),
            pl.BlockSpec((bb, 2), lambda i: (i, 0)),
            pl.BlockSpec((D, 128), lambda i: (0, 0)),
            pl.BlockSpec((D, 128), lambda i: (0, 0)),
            pl.BlockSpec((2, 128), lambda i: (0, 0)),
            pl.BlockSpec((1, 128), lambda i: (0, 0)),
            pl.BlockSpec((128, 64), lambda i: (0, 0)),
            pl.BlockSpec((1, 64), lambda i: (0, 0)),
            pl.BlockSpec((1, 64), lambda i: (0, 0)),
            pl.BlockSpec((1, 1), lambda i: (0, 0)),
        ],
        out_specs=pl.BlockSpec((bb, 1), lambda i: (i, 0)),
        out_shape=jax.ShapeDtypeStruct((B, 1), jnp.float32),
    )(xa, xt, num, w1a, w1t, w1n, b1, w2, b2, w3t, b3)


def kernel(artist, title, numerical_data, emb_artist, emb_title,
           W1, b1, W2, b2, W3, b3):
    vt = emb_title.shape[0]
    va = emb_artist.shape[0]
    ka = _repack_k(va)
    kt = _repack_k(vt)
    ai = artist.astype(jnp.int32)
    ti = title.astype(jnp.int32)
    artist_r = jnp.where(ai < ka, 2 * ai, 2 * (ai - ka) + 1).reshape(
        NW, NCHUNK, IPC)
    title_r = jnp.where(ti < kt, 2 * ti, 2 * (ti - kt) + 1).reshape(
        NW, NCHUNK, IPC)
    ttp = _repack(emb_title.T, vt)
    at_view, ttp = lax.optimization_barrier((emb_artist.T, ttp))
    pooled_t = _sc_pool(title_r, ttp.reshape(2 * kt, D))
    at64 = _repack(at_view, va).reshape(2 * ka, D)
    pooled_a = _sc_pool(artist_r, at64)
    return _mlp(pooled_a, pooled_t, numerical_data,
                W1[:D], W1[D:2 * D], W1[2 * D:], b1.reshape(1, 128),
                W2, b2.reshape(1, 64), W3.reshape(1, 64), b3.reshape(1, 1))


# speedup vs baseline: 1.8157x; 1.0577x over previous
import functools

import jax
import jax.numpy as jnp
from jax import lax
from jax.experimental import pallas as pl
from jax.experimental.pallas import tpu as pltpu
from jax.experimental.pallas import tpu_sc as plsc

B = 16384
L = 20
D = 64

NC = 2
NS = 16
NW = NC * NS
RPW = B // NW
CB = 4
IPC = CB * L
NCHUNK = RPW // CB
NBUF = 8


def _sc_pool_body(idx_ref, emb_ref, out_ref, idx_v, rows, out_v, sems):
    wid = lax.axis_index("s") * NC + lax.axis_index("c")
    pltpu.sync_copy(idx_ref.at[wid], idx_v)

    inv_l = jnp.float32(1.0 / L)

    for b in range(NBUF):
        pltpu.async_copy(emb_ref.at[idx_v.at[b]], rows.at[b], sems[b])

    def step(jj, carry):
        for b in range(NBUF):
            j = jj * NBUF + b
            pltpu.make_async_copy(
                emb_ref.at[idx_v.at[j]], rows.at[b], sems[b]).wait()
            for r in range(CB):
                row = j * CB + r
                for k in range(D // 16):
                    sl = pl.ds(k * 16, 16)
                    acc = rows[b, L * r, sl]
                    for q in range(1, L):
                        acc = acc + rows[b, L * r + q, sl]
                    out_v[row, sl] = acc * inv_l
            nxt = j + NBUF

            @pl.when(nxt < NCHUNK)
            def _fire():
                pltpu.async_copy(emb_ref.at[idx_v.at[nxt]], rows.at[b], sems[b])
        return carry

    lax.fori_loop(0, NCHUNK // NBUF, step, 0)
    pltpu.sync_copy(out_v, out_ref.at[pl.ds(wid * RPW, RPW)])


def _sc_pool(idx_r, table128):
    mesh = plsc.VectorSubcoreMesh(core_axis_name="c", subcore_axis_name="s",
                                  num_cores=NC, num_subcores=NS)
    return pl.kernel(
        _sc_pool_body,
        out_type=jax.ShapeDtypeStruct((B, D), jnp.float32),
        mesh=mesh,
        scratch_types=[
            pltpu.VMEM((NCHUNK, IPC), jnp.int32),
            pltpu.VMEM((NBUF, IPC, D), jnp.float32),
            pltpu.VMEM((RPW, D), jnp.float32),
            [pltpu.SemaphoreType.DMA] * NBUF,
        ],
        compiler_params=pltpu.CompilerParams(use_tc_tiling_on_sc=False),
    )(idx_r, table128)


def _tr_body(a_ref, b_ref, out_ref):
    out_ref[...] = jnp.concatenate([a_ref[...].T, b_ref[...].T], axis=1)


TB2 = 8192


def _repack_k(v):
    return TB2 * pl.cdiv(v // 2, TB2)


def _repack(table_t, v):
    k = _repack_k(v)
    nb = k // TB2
    grid = (nb,)
    bmax = pl.cdiv(v, TB2) - 1
    return pl.pallas_call(
        _tr_body,
        grid=grid,
        in_specs=[
            pl.BlockSpec((64, TB2), lambda i: (0, i)),
            pl.BlockSpec((64, TB2), lambda i: (0, jnp.minimum(i + nb, bmax))),
        ],
        out_specs=pl.BlockSpec((TB2, 128), lambda i: (i, 0)),
        out_shape=jax.ShapeDtypeStruct((k, 128), jnp.float32),
    )(table_t, table_t)


def _mlp_body(xa_ref, xt_ref, num_ref, w1a_ref, w1t_ref, w1n_ref, b1_ref,
              w2_ref, b2_ref, w3t_ref, b3_ref, out_ref):
    h1 = jnp.dot(xa_ref[...], w1a_ref[...], preferred_element_type=jnp.float32,
                 precision=jax.lax.Precision.HIGHEST)
    h1 = h1 + jnp.dot(xt_ref[...], w1t_ref[...],
                      preferred_element_type=jnp.float32,
                      precision=jax.lax.Precision.HIGHEST)
    num = num_ref[...]
    h1 = h1 + num[:, 0:1] * w1n_ref[0:1, :] + num[:, 1:2] * w1n_ref[1:2, :]
    h1 = jnp.maximum(h1 + b1_ref[...], 0.0)
    h2 = jnp.dot(h1, w2_ref[...], preferred_element_type=jnp.float32,
                 precision=jax.lax.Precision.HIGHEST)
    h2 = jnp.maximum(h2 + b2_ref[...], 0.0)
    out_ref[...] = jnp.sum(h2 * w3t_ref[...], axis=1, keepdims=True) + b3_ref[...]


def _mlp(xa, xt, num, w1a, w1t, w1n, b1, w2, b2, w3t, b3):
    bb = 2048
    grid = (B // bb,)
    return pl.pallas_call(
        _mlp_body,
        grid=grid,
        in_specs=[
            pl.BlockSpec((bb, D), lambda i: (i, 0)),
            pl.BlockSpec((bb, D), lambda i: (i, 0)),
            pl.BlockSpec((bb, 2), lambda i: (i, 0)),
            pl.BlockSpec((D, 128), lambda i: (0, 0)),
            pl.BlockSpec((D, 128), lambda i: (0, 0)),
            pl.BlockSpec((2, 128), lambda i: (0, 0)),
            pl.BlockSpec((1, 128), lambda i: (0, 0)),
            pl.BlockSpec((128, 64), lambda i: (0, 0)),
            pl.BlockSpec((1, 64), lambda i: (0, 0)),
            pl.BlockSpec((1, 64), lambda i: (0, 0)),
            pl.BlockSpec((1, 1), lambda i: (0, 0)),
        ],
        out_specs=pl.BlockSpec((bb, 1), lambda i: (i, 0)),
        out_shape=jax.ShapeDtypeStruct((B, 1), jnp.float32),
    )(xa, xt, num, w1a, w1t, w1n, b1, w2, b2, w3t, b3)


def kernel(artist, title, numerical_data, emb_artist, emb_title,
           W1, b1, W2, b2, W3, b3):
    vt = emb_title.shape[0]
    va = emb_artist.shape[0]
    ka = _repack_k(va)
    kt = _repack_k(vt)
    ai = artist.astype(jnp.int32)
    ti = title.astype(jnp.int32)
    artist_r = jnp.where(ai < ka, 2 * ai, 2 * (ai - ka) + 1).reshape(
        NW, NCHUNK, IPC)
    title_r = jnp.where(ti < kt, 2 * ti, 2 * (ti - kt) + 1).reshape(
        NW, NCHUNK, IPC)
    ttp = _repack(emb_title.T, vt)
    at_view, ttp = lax.optimization_barrier((emb_artist.T, ttp))
    pooled_t = _sc_pool(title_r, ttp.reshape(2 * kt, D))
    at64 = _repack(at_view, va).reshape(2 * ka, D)
    pooled_a = _sc_pool(artist_r, at64)
    return _mlp(pooled_a, pooled_t, numerical_data,
                W1[:D], W1[D:2 * D], W1[2 * D:], b1.reshape(1, 128),
                W2, b2.reshape(1, 64), W3.reshape(1, 64), b3.reshape(1, 1))


# speedup vs baseline: 1.8446x; 1.0159x over previous
import functools

import jax
import jax.numpy as jnp
from jax import lax
from jax.experimental import pallas as pl
from jax.experimental.pallas import tpu as pltpu
from jax.experimental.pallas import tpu_sc as plsc

B = 16384
L = 20
D = 64

NC = 2
NS = 16
NW = NC * NS
RPW = B // NW
CB = 4
IPC = CB * L
NCHUNK = RPW // CB
NBUF = 8


def _sc_pool_body(idx_ref, emb_ref, out_ref, idx_v, rows, out_v, sems):
    wid = lax.axis_index("s") * NC + lax.axis_index("c")
    pltpu.sync_copy(idx_ref.at[wid], idx_v)

    inv_l = jnp.float32(1.0 / L)

    for b in range(NBUF):
        pltpu.async_copy(emb_ref.at[idx_v.at[b]], rows.at[b], sems[b])

    def step(jj, carry):
        for b in range(NBUF):
            j = jj * NBUF + b
            pltpu.make_async_copy(
                emb_ref.at[idx_v.at[j]], rows.at[b], sems[b]).wait()
            for r in range(CB):
                row = j * CB + r
                for k in range(D // 16):
                    sl = pl.ds(k * 16, 16)
                    acc = rows[b, L * r, sl]
                    for q in range(1, L):
                        acc = acc + rows[b, L * r + q, sl]
                    out_v[row, sl] = acc * inv_l
            nxt = j + NBUF

            @pl.when(nxt < NCHUNK)
            def _fire():
                pltpu.async_copy(emb_ref.at[idx_v.at[nxt]], rows.at[b], sems[b])
        return carry

    lax.fori_loop(0, NCHUNK // NBUF, step, 0)
    pltpu.sync_copy(out_v, out_ref.at[pl.ds(wid * RPW, RPW)])


def _sc_pool(idx_r, table128):
    mesh = plsc.VectorSubcoreMesh(core_axis_name="c", subcore_axis_name="s",
                                  num_cores=NC, num_subcores=NS)
    return pl.kernel(
        _sc_pool_body,
        out_type=jax.ShapeDtypeStruct((B, D), jnp.float32),
        mesh=mesh,
        scratch_types=[
            pltpu.VMEM((NCHUNK, IPC), jnp.int32),
            pltpu.VMEM((NBUF, IPC, D), jnp.float32),
            pltpu.VMEM((RPW, D), jnp.float32),
            [pltpu.SemaphoreType.DMA] * NBUF,
        ],
        compiler_params=pltpu.CompilerParams(use_tc_tiling_on_sc=False),
    )(idx_r, table128)


def _tr_body(a_ref, b_ref, out_ref):
    out_ref[...] = jnp.concatenate([a_ref[...].T, b_ref[...].T], axis=1)


TB2 = 16384


def _repack_k(v):
    return TB2 * pl.cdiv(v // 2, TB2)


def _repack(table_t, v):
    k = _repack_k(v)
    nb = k // TB2
    grid = (nb,)
    bmax = pl.cdiv(v, TB2) - 1
    return pl.pallas_call(
        _tr_body,
        grid=grid,
        in_specs=[
            pl.BlockSpec((64, TB2), lambda i: (0, i)),
            pl.BlockSpec((64, TB2), lambda i: (0, jnp.minimum(i + nb, bmax))),
        ],
        out_specs=pl.BlockSpec((TB2, 128), lambda i: (i, 0)),
        out_shape=jax.ShapeDtypeStruct((k, 128), jnp.float32),
    )(table_t, table_t)


def _mlp_body(xa_ref, xt_ref, num_ref, w1a_ref, w1t_ref, w1n_ref, b1_ref,
              w2_ref, b2_ref, w3t_ref, b3_ref, out_ref):
    h1 = jnp.dot(xa_ref[...], w1a_ref[...], preferred_element_type=jnp.float32,
                 precision=jax.lax.Precision.HIGHEST)
    h1 = h1 + jnp.dot(xt_ref[...], w1t_ref[...],
                      preferred_element_type=jnp.float32,
                      precision=jax.lax.Precision.HIGHEST)
    num = num_ref[...]
    h1 = h1 + num[:, 0:1] * w1n_ref[0:1, :] + num[:, 1:2] * w1n_ref[1:2, :]
    h1 = jnp.maximum(h1 + b1_ref[...], 0.0)
    h2 = jnp.dot(h1, w2_ref[...], preferred_element_type=jnp.float32,
                 precision=jax.lax.Precision.HIGHEST)
    h2 = jnp.maximum(h2 + b2_ref[...], 0.0)
    out_ref[...] = jnp.sum(h2 * w3t_ref[...], axis=1, keepdims=True) + b3_ref[...]


def _mlp(xa, xt, num, w1a, w1t, w1n, b1, w2, b2, w3t, b3):
    bb = 2048
    grid = (B // bb,)
    return pl.pallas_call(
        _mlp_body,
        grid=grid,
        in_specs=[
            pl.BlockSpec((bb, D), lambda i: (i, 0)),
            pl.BlockSpec((bb, D), lambda i: (i, 0)),
            pl.BlockSpec((bb, 2), lambda i: (i, 0)),
            pl.BlockSpec((D, 128), lambda i: (0, 0)),
            pl.BlockSpec((D, 128), lambda i: (0, 0)),
            pl.BlockSpec((2, 128), lambda i: (0, 0)),
            pl.BlockSpec((1, 128), lambda i: (0, 0)),
            pl.BlockSpec((128, 64), lambda i: (0, 0)),
            pl.BlockSpec((1, 64), lambda i: (0, 0)),
            pl.BlockSpec((1, 64), lambda i: (0, 0)),
            pl.BlockSpec((1, 1), lambda i: (0, 0)),
        ],
        out_specs=pl.BlockSpec((bb, 1), lambda i: (i, 0)),
        out_shape=jax.ShapeDtypeStruct((B, 1), jnp.float32),
    )(xa, xt, num, w1a, w1t, w1n, b1, w2, b2, w3t, b3)


def kernel(artist, title, numerical_data, emb_artist, emb_title,
           W1, b1, W2, b2, W3, b3):
    vt = emb_title.shape[0]
    va = emb_artist.shape[0]
    ka = _repack_k(va)
    kt = _repack_k(vt)
    ai = artist.astype(jnp.int32)
    ti = title.astype(jnp.int32)
    artist_r = jnp.where(ai < ka, 2 * ai, 2 * (ai - ka) + 1).reshape(
        NW, NCHUNK, IPC)
    title_r = jnp.where(ti < kt, 2 * ti, 2 * (ti - kt) + 1).reshape(
        NW, NCHUNK, IPC)
    ttp = _repack(emb_title.T, vt)
    at_view, ttp = lax.optimization_barrier((emb_artist.T, ttp))
    pooled_t = _sc_pool(title_r, ttp.reshape(2 * kt, D))
    at64 = _repack(at_view, va).reshape(2 * ka, D)
    pooled_a = _sc_pool(artist_r, at64)
    return _mlp(pooled_a, pooled_t, numerical_data,
                W1[:D], W1[D:2 * D], W1[2 * D:], b1.reshape(1, 128),
                W2, b2.reshape(1, 64), W3.reshape(1, 64), b3.reshape(1, 1))


# speedup vs baseline: 1.9285x; 1.0455x over previous
import functools

import jax
import jax.numpy as jnp
from jax import lax
from jax.experimental import pallas as pl
from jax.experimental.pallas import tpu as pltpu
from jax.experimental.pallas import tpu_sc as plsc

B = 16384
L = 20
D = 64

NC = 2
NS = 16
NW = NC * NS
RPW = B // NW
CB = 4
IPC = CB * L
NCHUNK = RPW // CB
NBUF = 8


def _sc_pool_body(idx_ref, emb_ref, out_ref, idx_v, rows, out_v, sems):
    wid = lax.axis_index("s") * NC + lax.axis_index("c")
    pltpu.sync_copy(idx_ref.at[wid], idx_v)

    inv_l = jnp.float32(1.0 / L)

    for b in range(NBUF):
        pltpu.async_copy(emb_ref.at[idx_v.at[b]], rows.at[b], sems[b])

    def step(jj, carry):
        for b in range(NBUF):
            j = jj * NBUF + b
            pltpu.make_async_copy(
                emb_ref.at[idx_v.at[j]], rows.at[b], sems[b]).wait()
            for r in range(CB):
                row = j * CB + r
                for k in range(D // 16):
                    sl = pl.ds(k * 16, 16)
                    acc = rows[b, L * r, sl]
                    for q in range(1, L):
                        acc = acc + rows[b, L * r + q, sl]
                    out_v[row, sl] = acc * inv_l
            nxt = j + NBUF

            @pl.when(nxt < NCHUNK)
            def _fire():
                pltpu.async_copy(emb_ref.at[idx_v.at[nxt]], rows.at[b], sems[b])
        return carry

    lax.fori_loop(0, NCHUNK // NBUF, step, 0)
    pltpu.sync_copy(out_v, out_ref.at[pl.ds(wid * RPW, RPW)])


def _sc_pool(idx_r, table128):
    mesh = plsc.VectorSubcoreMesh(core_axis_name="c", subcore_axis_name="s",
                                  num_cores=NC, num_subcores=NS)
    return pl.kernel(
        _sc_pool_body,
        out_type=jax.ShapeDtypeStruct((B, D), jnp.float32),
        mesh=mesh,
        scratch_types=[
            pltpu.VMEM((NCHUNK, IPC), jnp.int32),
            pltpu.VMEM((NBUF, IPC, D), jnp.float32),
            pltpu.VMEM((RPW, D), jnp.float32),
            [pltpu.SemaphoreType.DMA] * NBUF,
        ],
        compiler_params=pltpu.CompilerParams(use_tc_tiling_on_sc=False),
    )(idx_r, table128)


def _tr_body(a_ref, b_ref, out_ref):
    out_ref[...] = jnp.concatenate([a_ref[...].T, b_ref[...].T], axis=1)


TB2 = 16384


def _repack_k(v):
    return TB2 * pl.cdiv(v // 2, TB2)


def _repack(table_t, v):
    k = _repack_k(v)
    nb = k // TB2
    grid = (nb,)
    bmax = pl.cdiv(v, TB2) - 1
    return pl.pallas_call(
        _tr_body,
        grid=grid,
        in_specs=[
            pl.BlockSpec((64, TB2), lambda i: (0, i)),
            pl.BlockSpec((64, TB2), lambda i: (0, jnp.minimum(i + nb, bmax))),
        ],
        out_specs=pl.BlockSpec((TB2, 128), lambda i: (i, 0)),
        out_shape=jax.ShapeDtypeStruct((k, 128), jnp.float32),
    )(table_t, table_t)


def _mlp_body(xa_ref, xt_ref, num_ref, w1_ref, b1_ref,
              w2_ref, b2_ref, w3_ref, b3_ref, out_ref):
    x = jnp.concatenate([xa_ref[...], xt_ref[...], num_ref[...]], axis=1)
    h1 = jnp.dot(x, w1_ref[...], preferred_element_type=jnp.float32)
    h1 = jnp.maximum(h1 + b1_ref[...], 0.0)
    h2 = jnp.dot(h1, w2_ref[...], preferred_element_type=jnp.float32)
    h2 = jnp.maximum(h2 + b2_ref[...], 0.0)
    out_ref[...] = jnp.dot(h2, w3_ref[...],
                           preferred_element_type=jnp.float32) + b3_ref[...]


def _mlp(xa, xt, num, w1, b1, w2, b2, w3, b3):
    bb = 2048
    grid = (B // bb,)
    return pl.pallas_call(
        _mlp_body,
        grid=grid,
        in_specs=[
            pl.BlockSpec((bb, D), lambda i: (i, 0)),
            pl.BlockSpec((bb, D), lambda i: (i, 0)),
            pl.BlockSpec((bb, 2), lambda i: (i, 0)),
            pl.BlockSpec((2 * D + 2, 128), lambda i: (0, 0)),
            pl.BlockSpec((1, 128), lambda i: (0, 0)),
            pl.BlockSpec((128, 64), lambda i: (0, 0)),
            pl.BlockSpec((1, 64), lambda i: (0, 0)),
            pl.BlockSpec((64, 1), lambda i: (0, 0)),
            pl.BlockSpec((1, 1), lambda i: (0, 0)),
        ],
        out_specs=pl.BlockSpec((bb, 1), lambda i: (i, 0)),
        out_shape=jax.ShapeDtypeStruct((B, 1), jnp.float32),
    )(xa, xt, num, w1, b1, w2, b2, w3, b3)


def kernel(artist, title, numerical_data, emb_artist, emb_title,
           W1, b1, W2, b2, W3, b3):
    vt = emb_title.shape[0]
    va = emb_artist.shape[0]
    ka = _repack_k(va)
    kt = _repack_k(vt)
    ai = artist.astype(jnp.int32)
    ti = title.astype(jnp.int32)
    artist_r = jnp.where(ai < ka, 2 * ai, 2 * (ai - ka) + 1).reshape(
        NW, NCHUNK, IPC)
    title_r = jnp.where(ti < kt, 2 * ti, 2 * (ti - kt) + 1).reshape(
        NW, NCHUNK, IPC)
    ttp = _repack(emb_title.T, vt)
    at_view, ttp = lax.optimization_barrier((emb_artist.T, ttp))
    pooled_t = _sc_pool(title_r, ttp.reshape(2 * kt, D))
    at64 = _repack(at_view, va).reshape(2 * ka, D)
    pooled_a = _sc_pool(artist_r, at64)
    return _mlp(pooled_a, pooled_t, numerical_data,
                W1, b1.reshape(1, 128),
                W2, b2.reshape(1, 64), W3, b3.reshape(1, 1))
